# baseline (speedup 1.0000x reference)
import jax
import jax.numpy as jnp
from jax.experimental import pallas as pl
from jax.experimental.pallas import tpu as pltpu

_BM = 200


def _fused_kernel(xf_ref, w_ref, b_ref, adj_ref, out_ref, xbf_ref):
    i = pl.program_id(0)

    @pl.when(i == 0)
    def _cast_x():
        xbf_ref[...] = xf_ref[...].astype(jnp.bfloat16)

    t = jnp.dot(
        adj_ref[...].astype(jnp.bfloat16),
        xbf_ref[...],
        preferred_element_type=jnp.float32,
    )
    h = jnp.dot(t, w_ref[...], preferred_element_type=jnp.float32)
    out_ref[...] = xf_ref[pl.ds(i * _BM, _BM), :] + b_ref[...] + h


def kernel(x, adj, W, b):
    n, d = x.shape
    b2 = b.reshape(1, d)
    ni = n // _BM
    embed = pl.pallas_call(
        _fused_kernel,
        grid=(ni,),
        in_specs=[
            pl.BlockSpec((n, d), lambda i: (0, 0)),
            pl.BlockSpec((d, d), lambda i: (0, 0)),
            pl.BlockSpec((1, d), lambda i: (0, 0)),
            pl.BlockSpec((_BM, n), lambda i: (i, 0)),
        ],
        out_specs=pl.BlockSpec((_BM, d), lambda i: (i, 0)),
        out_shape=jax.ShapeDtypeStruct((n, d), jnp.float32),
        scratch_shapes=[pltpu.VMEM((n, d), jnp.bfloat16)],
        compiler_params=pltpu.CompilerParams(
            dimension_semantics=("arbitrary",),
            vmem_limit_bytes=100 * 1024 * 1024,
        ),
    )(x, W, b2, adj)
    return (embed, jnp.array(0.0, dtype=jnp.float32))


# speedup vs baseline: 1.0194x; 1.0194x over previous
import jax
import jax.numpy as jnp
from jax.experimental import pallas as pl
from jax.experimental.pallas import tpu as pltpu

_BM = 400


def _fused_kernel(xf_ref, w_ref, b_ref, adj_ref, out_ref):
    i = pl.program_id(0)
    t = jnp.dot(
        adj_ref[...],
        xf_ref[...],
        preferred_element_type=jnp.float32,
    )
    h = jnp.dot(t, w_ref[...], preferred_element_type=jnp.float32)
    out_ref[...] = xf_ref[pl.ds(i * _BM, _BM), :] + b_ref[...] + h


def kernel(x, adj, W, b):
    n, d = x.shape
    b2 = b.reshape(1, d)
    ni = n // _BM
    embed = pl.pallas_call(
        _fused_kernel,
        grid=(ni,),
        in_specs=[
            pl.BlockSpec((n, d), lambda i: (0, 0)),
            pl.BlockSpec((d, d), lambda i: (0, 0)),
            pl.BlockSpec((1, d), lambda i: (0, 0)),
            pl.BlockSpec((_BM, n), lambda i: (i, 0)),
        ],
        out_specs=pl.BlockSpec((_BM, d), lambda i: (i, 0)),
        out_shape=jax.ShapeDtypeStruct((n, d), jnp.float32),
        compiler_params=pltpu.CompilerParams(
            dimension_semantics=("arbitrary",),
            vmem_limit_bytes=100 * 1024 * 1024,
        ),
    )(x, W, b2, adj)
    return (embed, jnp.array(0.0, dtype=jnp.float32))
